# baseline (speedup 1.0000x reference)
import jax
import jax.numpy as jnp
from jax.experimental import pallas as pl
from jax.experimental.pallas import tpu as pltpu

MEM_LEN = 16384
OUT_DIM = 2048
IN_DIM = 1024
ROW_BLOCK = 1024
NBLK = MEM_LEN // ROW_BLOCK


def _fused_body(x_ref, mean_ref, std_ref, w_ref, b_ref, meml_ref, memr_ref,
                out_ref, e_scr, acc_ref):
    i = pl.program_id(0)

    @pl.when(i == 0)
    def _encoder():
        x = x_ref[...]
        mean = mean_ref[...]
        std = std_ref[...]
        new = (x - mean) / (std + 1e-07)
        new = jnp.where(std == 0, jnp.zeros_like(new), new)
        logits = jnp.dot(new, w_ref[...], preferred_element_type=jnp.float32)
        logits = logits + b_ref[...]
        m = jnp.max(logits, axis=-1, keepdims=True)
        shifted = logits - m
        lse = jnp.log(jnp.sum(jnp.exp(shifted), axis=-1, keepdims=True))
        e_scr[...] = shifted - lse
        acc_ref[0, 0] = jnp.inf

    d = (jnp.sum(jnp.abs(meml_ref[...] - e_scr[:, :OUT_DIM // 2]), axis=1)
         + jnp.sum(jnp.abs(memr_ref[...] - e_scr[:, OUT_DIM // 2:]), axis=1))
    blk_min = jnp.min(d)
    acc_ref[0, 0] = jnp.minimum(acc_ref[0, 0], blk_min)

    @pl.when(i == pl.num_programs(0) - 1)
    def _emit():
        out_ref[0, 0] = acc_ref[0, 0]


@jax.jit
def kernel(x, mean, std, memory, W_enc, b_enc):
    mean2 = mean.reshape(1, IN_DIM)
    std2 = std.reshape(1, IN_DIM)
    b2 = b_enc.reshape(1, OUT_DIM)

    zero = lambda i: (0, 0)
    mem_idx = lambda i: (i, 0)

    out = pl.pallas_call(
        _fused_body,
        grid=(NBLK,),
        in_specs=[
            pl.BlockSpec((1, IN_DIM), zero),
            pl.BlockSpec((1, IN_DIM), zero),
            pl.BlockSpec((1, IN_DIM), zero),
            pl.BlockSpec((IN_DIM, OUT_DIM), zero),
            pl.BlockSpec((1, OUT_DIM), zero),
            pl.BlockSpec((ROW_BLOCK, OUT_DIM // 2), lambda i: (i, 0)),
            pl.BlockSpec((ROW_BLOCK, OUT_DIM // 2), lambda i: (i, 1)),
        ],
        out_specs=pl.BlockSpec(memory_space=pltpu.SMEM),
        out_shape=jax.ShapeDtypeStruct((1, 1), jnp.float32),
        scratch_shapes=[
            pltpu.VMEM((1, OUT_DIM), jnp.float32),
            pltpu.SMEM((1, 1), jnp.float32),
        ],
    )(x, mean2, std2, W_enc, b2, memory, memory)
    return out[0, 0]


# speedup vs baseline: 1.0372x; 1.0372x over previous
import jax
import jax.numpy as jnp
from jax.experimental import pallas as pl
from jax.experimental.pallas import tpu as pltpu

MEM_LEN = 16384
OUT_DIM = 2048
IN_DIM = 1024
ROW_BLOCK = 1024
NBLK = MEM_LEN // ROW_BLOCK


def _fused_body(x_ref, mean_ref, std_ref, w_ref, b_ref, mem_ref,
                out_ref, e_scr, acc_ref):
    i = pl.program_id(0)

    @pl.when(i == 0)
    def _encoder():
        x = x_ref[...]
        mean = mean_ref[...]
        std = std_ref[...]
        new = (x - mean) / (std + 1e-07)
        new = jnp.where(std == 0, jnp.zeros_like(new), new)
        logits = jnp.dot(new, w_ref[...], preferred_element_type=jnp.float32)
        logits = logits + b_ref[...]
        m = jnp.max(logits, axis=-1, keepdims=True)
        shifted = logits - m
        lse = jnp.log(jnp.sum(jnp.exp(shifted), axis=-1, keepdims=True))
        e_scr[...] = shifted - lse
        acc_ref[0, 0] = jnp.inf

    d = jnp.sum(jnp.abs(mem_ref[...] - e_scr[...]), axis=1)
    blk_min = jnp.min(d)
    acc_ref[0, 0] = jnp.minimum(acc_ref[0, 0], blk_min)

    @pl.when(i == pl.num_programs(0) - 1)
    def _emit():
        out_ref[0, 0] = acc_ref[0, 0]


@jax.jit
def kernel(x, mean, std, memory, W_enc, b_enc):
    mean2 = mean.reshape(1, IN_DIM)
    std2 = std.reshape(1, IN_DIM)
    b2 = b_enc.reshape(1, OUT_DIM)

    zero = lambda i: (0, 0)
    mem_idx = lambda i: (i, 0)

    out = pl.pallas_call(
        _fused_body,
        grid=(NBLK,),
        in_specs=[
            pl.BlockSpec((1, IN_DIM), zero),
            pl.BlockSpec((1, IN_DIM), zero),
            pl.BlockSpec((1, IN_DIM), zero),
            pl.BlockSpec((IN_DIM, OUT_DIM), zero),
            pl.BlockSpec((1, OUT_DIM), zero),
            pl.BlockSpec((ROW_BLOCK, OUT_DIM), mem_idx),
        ],
        out_specs=pl.BlockSpec(memory_space=pltpu.SMEM),
        out_shape=jax.ShapeDtypeStruct((1, 1), jnp.float32),
        scratch_shapes=[
            pltpu.VMEM((1, OUT_DIM), jnp.float32),
            pltpu.SMEM((1, 1), jnp.float32),
        ],
    )(x, mean2, std2, W_enc, b2, memory)
    return out[0, 0]
